# baseline (speedup 1.0000x reference)
import functools

import jax
import jax.numpy as jnp
from jax import lax
from jax.experimental import pallas as pl
from jax.experimental.pallas import tpu as pltpu
from jax.experimental.pallas import tpu_sc as plsc

N = 10000
E = 320000
D = 128
DFF = 256
EPS = 1e-5

NC = 2
NS = 16
NW = NC * NS
CH = 96
NCH = 105
E_PAD = NW * NCH * CH
NGRP = NCH // 3
RPT = 632
NROW = NS * RPT
DEG_CH = 640
NPAD = NS * DEG_CH

_mesh = plsc.VectorSubcoreMesh(core_axis_name="c", subcore_axis_name="s")


@functools.partial(
    pl.kernel,
    out_type=jax.ShapeDtypeStruct((NC * NPAD,), jnp.float32),
    mesh=_mesh,
    scratch_types=[
        pltpu.VMEM((2, 3, CH), jnp.int32),
        pltpu.VMEM((CH,), jnp.float32),
        pltpu.SemaphoreType.DMA,
        pltpu.SemaphoreType.DMA,
        pltpu.VMEM_SHARED((NPAD,), jnp.float32),
    ],
)
def _deg_kernel(dst_hbm, zeros_hbm, out_hbm, didx, ones_v, isem, ssem, acc):
    c = lax.axis_index("c")
    s = lax.axis_index("s")
    wid = s * NC + c
    ebase = wid * NCH * CH

    def fill(i, carry):
        ones_v[pl.ds(i * 16, 16)] = jnp.ones((16,), jnp.float32)
        return carry

    lax.fori_loop(0, CH // 16, fill, 0)

    for k in range(3):
        pltpu.sync_copy(dst_hbm.at[pl.ds(ebase + k * CH, CH)], didx.at[0, k])
    pltpu.sync_copy(zeros_hbm, acc.at[pl.ds(s * DEG_CH, DEG_CH)])
    plsc.subcore_barrier()

    def group(g, carry):
        slot = lax.rem(g, 2)
        nslot = 1 - slot

        @pl.when(g < NGRP - 1)
        def _():
            nb = ebase + (3 * g + 3) * CH
            for k in range(3):
                pltpu.async_copy(dst_hbm.at[pl.ds(nb + k * CH, CH)],
                                 didx.at[nslot, k], isem)

        descs = []
        for k in range(3):
            descs.append(pltpu.async_copy(
                ones_v, acc.at[didx.at[slot, k]], ssem, add=True))

        @pl.when(g < NGRP - 1)
        def _():
            for _j in range(3):
                pltpu.make_async_copy(dst_hbm.at[pl.ds(ebase, CH)],
                                      didx.at[nslot, 0], isem).wait()

        for d in descs:
            d.wait()
        return carry

    lax.fori_loop(0, NGRP, group, 0)
    plsc.subcore_barrier()
    pltpu.sync_copy(acc.at[pl.ds(s * DEG_CH, DEG_CH)],
                    out_hbm.at[pl.ds(c * NPAD + s * DEG_CH, DEG_CH)])


@functools.partial(
    pl.kernel,
    out_type=jax.ShapeDtypeStruct((NC, NROW, D), jnp.float32),
    mesh=_mesh,
    scratch_types=[
        pltpu.VMEM((2, 3, CH), jnp.int32),
        pltpu.VMEM((2, 3, CH), jnp.int32),
        pltpu.VMEM((3, CH, D), jnp.float32),
        pltpu.SemaphoreType.DMA,
        pltpu.SemaphoreType.DMA,
        pltpu.SemaphoreType.DMA,
        pltpu.SemaphoreType.DMA,
        pltpu.SemaphoreType.DMA,
        pltpu.SemaphoreType.DMA,
        pltpu.SemaphoreType.DMA,
        pltpu.VMEM_SHARED((NROW, D), jnp.float32),
    ],
)
def _agg_kernel(y_hbm, src_hbm, dst_hbm, zeros_hbm, out_hbm,
                sidx, didx, rows, isem, gs0, gs1, gs2, ss0, ss1, ss2, acc):
    c = lax.axis_index("c")
    s = lax.axis_index("s")
    wid = s * NC + c
    ebase = wid * NCH * CH
    gsems = [gs0, gs1, gs2]
    ssems = [ss0, ss1, ss2]

    for k in range(3):
        pltpu.sync_copy(src_hbm.at[pl.ds(ebase + k * CH, CH)], sidx.at[0, k])
        pltpu.sync_copy(dst_hbm.at[pl.ds(ebase + k * CH, CH)], didx.at[0, k])
    pltpu.async_copy(y_hbm.at[sidx.at[0, 0]], rows.at[0], gs0)
    pltpu.async_copy(y_hbm.at[sidx.at[0, 1]], rows.at[1], gs1)
    pltpu.sync_copy(zeros_hbm, acc.at[pl.ds(s * RPT, RPT)])
    plsc.subcore_barrier()

    def group(g, carry):
        slot = lax.rem(g, 2)
        nslot = 1 - slot

        @pl.when(g < NGRP - 1)
        def _():
            nb = ebase + (3 * g + 3) * CH
            for k in range(3):
                pltpu.async_copy(src_hbm.at[pl.ds(nb + k * CH, CH)],
                                 sidx.at[nslot, k], isem)
                pltpu.async_copy(dst_hbm.at[pl.ds(nb + k * CH, CH)],
                                 didx.at[nslot, k], isem)

        for k in range(3):
            pltpu.make_async_copy(y_hbm.at[sidx.at[slot, k]],
                                  rows.at[k], gsems[k]).wait()
            pltpu.async_copy(rows.at[k], acc.at[didx.at[slot, k]],
                             ssems[k], add=True)
            fb = (k + 2) % 3
            if k == 0:
                @pl.when(g > 0)
                def _():
                    pltpu.make_async_copy(rows.at[fb], acc.at[didx.at[slot, k]],
                                          ssems[fb]).wait()
                pltpu.async_copy(y_hbm.at[sidx.at[slot, 2]], rows.at[2], gs2)
            else:
                pltpu.make_async_copy(rows.at[fb], acc.at[didx.at[slot, k]],
                                      ssems[fb]).wait()

                @pl.when(g < NGRP - 1)
                def _():
                    if k == 1:
                        for _j in range(6):
                            pltpu.make_async_copy(
                                src_hbm.at[pl.ds(ebase, CH)],
                                sidx.at[nslot, 0], isem).wait()
                    pltpu.async_copy(y_hbm.at[sidx.at[nslot, k - 1]],
                                     rows.at[fb], gsems[fb])

        return carry

    lax.fori_loop(0, NGRP, group, 0)
    pltpu.make_async_copy(rows.at[2], acc.at[didx.at[0, 2]], ss2).wait()
    plsc.subcore_barrier()
    pltpu.sync_copy(acc.at[pl.ds(s * RPT, RPT)],
                    out_hbm.at[c, pl.ds(s * RPT, RPT)])


def _scale_body(deg_ref, x_ref, y_ref, dinv_ref):
    dinv = lax.rsqrt(deg_ref[...])
    dinv_ref[...] = dinv
    y_ref[...] = x_ref[...] * dinv


_scale_call = pl.pallas_call(
    _scale_body,
    out_shape=(
        jax.ShapeDtypeStruct((N, D), jnp.float32),
        jax.ShapeDtypeStruct((N, 1), jnp.float32),
    ),
)


def _bn(h, g, b):
    mu = jnp.mean(h, axis=0, keepdims=True)
    var = jnp.mean((h - mu) ** 2, axis=0, keepdims=True)
    return (h - mu) * lax.rsqrt(var + EPS) * g + b


def _dense_body(part_ref, y_ref, dinv_ref, x_ref, W_ref, b_ref,
                bn_g_ref, bn_b_ref, bn1_g_ref, bn1_b_ref, W1_ref, b1_ref,
                W2_ref, b2_ref, bn2_g_ref, bn2_b_ref, out_ref):
    p0 = lax.slice(part_ref[0], (0, 0), (N, D))
    p1 = lax.slice(part_ref[1], (0, 0), (N, D))
    agg = (p0 + p1 + y_ref[...]) * dinv_ref[...]
    conv = jnp.dot(agg, W_ref[...], preferred_element_type=jnp.float32)
    conv = conv + b_ref[...]
    h = _bn(conv, bn_g_ref[...], bn_b_ref[...])
    h = jnp.maximum(h, 0.0) + x_ref[...]
    z = _bn(h, bn1_g_ref[...], bn1_b_ref[...])
    z = jnp.dot(z, W1_ref[...], preferred_element_type=jnp.float32) + b1_ref[...]
    z = jnp.maximum(z, 0.0)
    z = jnp.dot(z, W2_ref[...], preferred_element_type=jnp.float32) + b2_ref[...]
    h = h + z
    out_ref[...] = _bn(h, bn2_g_ref[...], bn2_b_ref[...])


_dense_call = pl.pallas_call(
    _dense_body,
    out_shape=jax.ShapeDtypeStruct((N, D), jnp.float32),
)


def kernel(x, edge_index, W, b, bn_g, bn_b, bn1_g, bn1_b,
           W1, b1, W2, b2, bn2_g, bn2_b):
    pad = E_PAD - E
    pad_src = jnp.arange(pad, dtype=jnp.int32) % N
    src = jnp.concatenate([edge_index[0], pad_src])
    pad_dst = N + jnp.arange(pad, dtype=jnp.int32) % (NROW - N)
    dst = jnp.concatenate([edge_index[1], pad_dst])
    zeros_vec = jnp.zeros((DEG_CH,), jnp.float32)
    zeros_mat = jnp.zeros((RPT, D), jnp.float32)

    degp = _deg_kernel(dst, zeros_vec)
    deg_col = (degp[:N] + degp[NPAD:NPAD + N] + 1.0).reshape(N, 1)
    y, dinv = _scale_call(deg_col, x)
    part = _agg_kernel(y, src, dst, zeros_mat)
    out = _dense_call(
        part, y, dinv, x, W, b.reshape(1, D),
        bn_g.reshape(1, D), bn_b.reshape(1, D),
        bn1_g.reshape(1, D), bn1_b.reshape(1, D),
        W1, b1.reshape(1, DFF), W2, b2.reshape(1, D),
        bn2_g.reshape(1, D), bn2_b.reshape(1, D),
    )
    return out


# speedup vs baseline: 1.0881x; 1.0881x over previous
import functools

import jax
import jax.numpy as jnp
from jax import lax
from jax.experimental import pallas as pl
from jax.experimental.pallas import tpu as pltpu
from jax.experimental.pallas import tpu_sc as plsc

N = 10000
E = 320000
D = 128
DFF = 256
EPS = 1e-5

NC = 2
NS = 16
NW = NC * NS
CH = 64
NCH = 156
EPW = NCH * CH
NGRP = NCH // 3
NEXTRA = E // CH - NW * NCH
RPT = 632
NROW = NS * RPT
DEG_CH = 640
NPAD = NS * DEG_CH

_mesh = plsc.VectorSubcoreMesh(core_axis_name="c", subcore_axis_name="s")


@functools.partial(
    pl.kernel,
    out_type=jax.ShapeDtypeStruct((NC * NPAD,), jnp.float32),
    mesh=_mesh,
    scratch_types=[
        pltpu.VMEM((EPW,), jnp.int32),
        pltpu.VMEM((CH,), jnp.float32),
        pltpu.SemaphoreType.DMA,
        pltpu.VMEM_SHARED((NPAD,), jnp.float32),
    ],
)
def _deg_kernel(ef_hbm, zeros_hbm, out_hbm, didx, ones_v, ssem, acc):
    c = lax.axis_index("c")
    s = lax.axis_index("s")
    wid = s * NC + c
    ebase = E + wid * EPW

    def fill(i, carry):
        ones_v[pl.ds(i * 16, 16)] = jnp.ones((16,), jnp.float32)
        return carry

    lax.fori_loop(0, CH // 16, fill, 0)

    pltpu.sync_copy(ef_hbm.at[pl.ds(ebase, EPW)], didx)
    pltpu.sync_copy(zeros_hbm, acc.at[pl.ds(s * DEG_CH, DEG_CH)])
    plsc.subcore_barrier()

    def group(g, carry):
        descs = []
        for k in range(3):
            descs.append(pltpu.async_copy(
                ones_v, acc.at[didx.at[pl.ds((3 * g + k) * CH, CH)]],
                ssem, add=True))
        for d in descs:
            d.wait()
        return carry

    lax.fori_loop(0, NGRP, group, 0)

    @pl.when(wid < NEXTRA)
    def _():
        eb = E + (NW * NCH + wid) * CH
        pltpu.sync_copy(ef_hbm.at[pl.ds(eb, CH)], didx.at[pl.ds(0, CH)])
        pltpu.sync_copy(ones_v, acc.at[didx.at[pl.ds(0, CH)]], add=True)

    plsc.subcore_barrier()
    pltpu.sync_copy(acc.at[pl.ds(s * DEG_CH, DEG_CH)],
                    out_hbm.at[pl.ds(c * NPAD + s * DEG_CH, DEG_CH)])


@functools.partial(
    pl.kernel,
    out_type=jax.ShapeDtypeStruct((NC, NROW, D), jnp.float32),
    mesh=_mesh,
    scratch_types=[
        pltpu.VMEM((EPW,), jnp.int32),
        pltpu.VMEM((EPW,), jnp.int32),
        pltpu.VMEM((3, CH, D), jnp.float32),
        pltpu.SemaphoreType.DMA,
        pltpu.SemaphoreType.DMA,
        pltpu.SemaphoreType.DMA,
        pltpu.SemaphoreType.DMA,
        pltpu.SemaphoreType.DMA,
        pltpu.SemaphoreType.DMA,
        pltpu.VMEM_SHARED((NROW, D), jnp.float32),
    ],
)
def _agg_kernel(y_hbm, ef_hbm, zeros_hbm, out_hbm,
                sidx, didx, rows, gs0, gs1, gs2, ss0, ss1, ss2, acc):
    c = lax.axis_index("c")
    s = lax.axis_index("s")
    wid = s * NC + c
    ebase = wid * EPW
    gsems = [gs0, gs1, gs2]
    ssems = [ss0, ss1, ss2]

    pltpu.sync_copy(ef_hbm.at[pl.ds(ebase, EPW)], sidx)
    pltpu.sync_copy(ef_hbm.at[pl.ds(E + ebase, EPW)], didx)
    pltpu.async_copy(y_hbm.at[sidx.at[pl.ds(0, CH)]], rows.at[0], gs0)
    pltpu.async_copy(y_hbm.at[sidx.at[pl.ds(CH, CH)]], rows.at[1], gs1)
    pltpu.sync_copy(zeros_hbm, acc.at[pl.ds(s * RPT, RPT)])
    plsc.subcore_barrier()

    def group(g, carry):
        for k in range(3):
            ck = (3 * g + k) * CH
            pltpu.make_async_copy(y_hbm.at[sidx.at[pl.ds(0, CH)]],
                                  rows.at[k], gsems[k]).wait()
            pltpu.async_copy(rows.at[k], acc.at[didx.at[pl.ds(ck, CH)]],
                             ssems[k], add=True)
            fb = (k + 2) % 3
            nk = (3 * g + k + 2) * CH
            if k == 0:
                @pl.when(g > 0)
                def _():
                    pltpu.make_async_copy(rows.at[fb], acc.at[didx.at[pl.ds(0, CH)]],
                                          ssems[fb]).wait()
                pltpu.async_copy(y_hbm.at[sidx.at[pl.ds(nk, CH)]],
                                 rows.at[fb], gsems[fb])
            else:
                pltpu.make_async_copy(rows.at[fb], acc.at[didx.at[pl.ds(0, CH)]],
                                      ssems[fb]).wait()

                @pl.when(g < NGRP - 1)
                def _():
                    pltpu.async_copy(y_hbm.at[sidx.at[pl.ds(nk, CH)]],
                                     rows.at[fb], gsems[fb])

        return carry

    lax.fori_loop(0, NGRP, group, 0)
    pltpu.make_async_copy(rows.at[2], acc.at[didx.at[pl.ds(0, CH)]], ss2).wait()

    @pl.when(wid < NEXTRA)
    def _():
        eb = (NW * NCH + wid) * CH
        pltpu.sync_copy(ef_hbm.at[pl.ds(eb, CH)], sidx.at[pl.ds(0, CH)])
        pltpu.sync_copy(ef_hbm.at[pl.ds(E + eb, CH)], didx.at[pl.ds(0, CH)])
        pltpu.async_copy(y_hbm.at[sidx.at[pl.ds(0, CH)]], rows.at[0], gs0).wait()
        pltpu.sync_copy(rows.at[0], acc.at[didx.at[pl.ds(0, CH)]], add=True)

    plsc.subcore_barrier()
    pltpu.sync_copy(acc.at[pl.ds(s * RPT, RPT)],
                    out_hbm.at[c, pl.ds(s * RPT, RPT)])


def _scale_body(deg_ref, x_ref, y_ref, dinv_ref):
    dinv = lax.rsqrt(deg_ref[...])
    dinv_ref[...] = dinv
    y_ref[...] = x_ref[...] * dinv


_scale_call = pl.pallas_call(
    _scale_body,
    out_shape=(
        jax.ShapeDtypeStruct((N, D), jnp.float32),
        jax.ShapeDtypeStruct((N, 1), jnp.float32),
    ),
)


def _bn(h, g, b):
    mu = jnp.mean(h, axis=0, keepdims=True)
    var = jnp.mean((h - mu) ** 2, axis=0, keepdims=True)
    return (h - mu) * lax.rsqrt(var + EPS) * g + b


def _dense_body(part_ref, y_ref, dinv_ref, x_ref, W_ref, b_ref,
                bn_g_ref, bn_b_ref, bn1_g_ref, bn1_b_ref, W1_ref, b1_ref,
                W2_ref, b2_ref, bn2_g_ref, bn2_b_ref, out_ref):
    p0 = lax.slice(part_ref[0], (0, 0), (N, D))
    p1 = lax.slice(part_ref[1], (0, 0), (N, D))
    agg = (p0 + p1 + y_ref[...]) * dinv_ref[...]
    conv = jnp.dot(agg, W_ref[...], preferred_element_type=jnp.float32)
    conv = conv + b_ref[...]
    h = _bn(conv, bn_g_ref[...], bn_b_ref[...])
    h = jnp.maximum(h, 0.0) + x_ref[...]
    z = _bn(h, bn1_g_ref[...], bn1_b_ref[...])
    z = jnp.dot(z, W1_ref[...], preferred_element_type=jnp.float32) + b1_ref[...]
    z = jnp.maximum(z, 0.0)
    z = jnp.dot(z, W2_ref[...], preferred_element_type=jnp.float32) + b2_ref[...]
    h = h + z
    out_ref[...] = _bn(h, bn2_g_ref[...], bn2_b_ref[...])


_dense_call = pl.pallas_call(
    _dense_body,
    out_shape=jax.ShapeDtypeStruct((N, D), jnp.float32),
)


def kernel(x, edge_index, W, b, bn_g, bn_b, bn1_g, bn1_b,
           W1, b1, W2, b2, bn2_g, bn2_b):
    ef = edge_index.reshape(2 * E)
    zeros_vec = jnp.zeros((DEG_CH,), jnp.float32)
    zeros_mat = jnp.zeros((RPT, D), jnp.float32)

    degp = _deg_kernel(ef, zeros_vec)
    deg_col = (degp[:N] + degp[NPAD:NPAD + N] + 1.0).reshape(N, 1)
    y, dinv = _scale_call(deg_col, x)
    part = _agg_kernel(y, ef, zeros_mat)
    out = _dense_call(
        part, y, dinv, x, W, b.reshape(1, D),
        bn_g.reshape(1, D), bn_b.reshape(1, D),
        bn1_g.reshape(1, D), bn1_b.reshape(1, D),
        W1, b1.reshape(1, DFF), W2, b2.reshape(1, D),
        bn2_g.reshape(1, D), bn2_b.reshape(1, D),
    )
    return out


# speedup vs baseline: 1.1079x; 1.0182x over previous
import functools

import jax
import jax.numpy as jnp
from jax import lax
from jax.experimental import pallas as pl
from jax.experimental.pallas import tpu as pltpu
from jax.experimental.pallas import tpu_sc as plsc

N = 10000
E = 320000
D = 128
DFF = 256
EPS = 1e-5

NC = 2
NS = 16
NW = NC * NS
CH = 64
NCH = 156
EPW = NCH * CH
NGRP = NCH // 3
NEXTRA = E // CH - NW * NCH
RPT = 632
NROW = NS * RPT
DEG_CH = 640
NPAD = NS * DEG_CH

_mesh = plsc.VectorSubcoreMesh(core_axis_name="c", subcore_axis_name="s")


@functools.partial(
    pl.kernel,
    out_type=jax.ShapeDtypeStruct((NC * NPAD,), jnp.float32),
    mesh=_mesh,
    scratch_types=[
        pltpu.VMEM((EPW,), jnp.int32),
        pltpu.VMEM((CH,), jnp.float32),
        pltpu.SemaphoreType.DMA,
        pltpu.VMEM_SHARED((NPAD,), jnp.float32),
    ],
)
def _deg_kernel(ef_hbm, zeros_hbm, out_hbm, didx, ones_v, ssem, acc):
    c = lax.axis_index("c")
    s = lax.axis_index("s")
    wid = s * NC + c
    ebase = wid * EPW

    def fill(i, carry):
        ones_v[pl.ds(i * 16, 16)] = jnp.ones((16,), jnp.float32)
        return carry

    lax.fori_loop(0, CH // 16, fill, 0)

    pltpu.sync_copy(ef_hbm.at[1, pl.ds(ebase, EPW)], didx)
    pltpu.sync_copy(zeros_hbm, acc.at[pl.ds(s * DEG_CH, DEG_CH)])
    plsc.subcore_barrier()

    def group(g, carry):
        descs = []
        for k in range(3):
            descs.append(pltpu.async_copy(
                ones_v, acc.at[didx.at[pl.ds((3 * g + k) * CH, CH)]],
                ssem, add=True))
        for d in descs:
            d.wait()
        return carry

    lax.fori_loop(0, NGRP, group, 0)

    @pl.when(wid < NEXTRA)
    def _():
        eb = (NW * NCH + wid) * CH
        pltpu.sync_copy(ef_hbm.at[1, pl.ds(eb, CH)], didx.at[pl.ds(0, CH)])
        pltpu.sync_copy(ones_v, acc.at[didx.at[pl.ds(0, CH)]], add=True)

    plsc.subcore_barrier()
    pltpu.sync_copy(acc.at[pl.ds(s * DEG_CH, DEG_CH)],
                    out_hbm.at[pl.ds(c * NPAD + s * DEG_CH, DEG_CH)])


@functools.partial(
    pl.kernel,
    out_type=jax.ShapeDtypeStruct((NC, NROW, D), jnp.float32),
    mesh=_mesh,
    scratch_types=[
        pltpu.VMEM((EPW,), jnp.int32),
        pltpu.VMEM((EPW,), jnp.int32),
        pltpu.VMEM((3, CH, D), jnp.float32),
        pltpu.SemaphoreType.DMA,
        pltpu.SemaphoreType.DMA,
        pltpu.SemaphoreType.DMA,
        pltpu.SemaphoreType.DMA,
        pltpu.SemaphoreType.DMA,
        pltpu.SemaphoreType.DMA,
        pltpu.VMEM_SHARED((NROW, D), jnp.float32),
    ],
)
def _agg_kernel(y_hbm, ef_hbm, zeros_hbm, out_hbm,
                sidx, didx, rows, gs0, gs1, gs2, ss0, ss1, ss2, acc):
    c = lax.axis_index("c")
    s = lax.axis_index("s")
    wid = s * NC + c
    ebase = wid * EPW
    gsems = [gs0, gs1, gs2]
    ssems = [ss0, ss1, ss2]

    pltpu.sync_copy(ef_hbm.at[0, pl.ds(ebase, EPW)], sidx)
    pltpu.sync_copy(ef_hbm.at[1, pl.ds(ebase, EPW)], didx)
    pltpu.async_copy(y_hbm.at[sidx.at[pl.ds(0, CH)]], rows.at[0], gs0)
    pltpu.async_copy(y_hbm.at[sidx.at[pl.ds(CH, CH)]], rows.at[1], gs1)
    pltpu.sync_copy(zeros_hbm, acc.at[pl.ds(s * RPT, RPT)])
    plsc.subcore_barrier()

    def group(g, carry):
        for k in range(3):
            ck = (3 * g + k) * CH
            pltpu.make_async_copy(y_hbm.at[sidx.at[pl.ds(0, CH)]],
                                  rows.at[k], gsems[k]).wait()
            pltpu.async_copy(rows.at[k], acc.at[didx.at[pl.ds(ck, CH)]],
                             ssems[k], add=True)
            fb = (k + 2) % 3
            nk = (3 * g + k + 2) * CH
            if k == 0:
                @pl.when(g > 0)
                def _():
                    pltpu.make_async_copy(rows.at[fb], acc.at[didx.at[pl.ds(0, CH)]],
                                          ssems[fb]).wait()
                pltpu.async_copy(y_hbm.at[sidx.at[pl.ds(nk, CH)]],
                                 rows.at[fb], gsems[fb])
            else:
                pltpu.make_async_copy(rows.at[fb], acc.at[didx.at[pl.ds(0, CH)]],
                                      ssems[fb]).wait()

                @pl.when(g < NGRP - 1)
                def _():
                    pltpu.async_copy(y_hbm.at[sidx.at[pl.ds(nk, CH)]],
                                     rows.at[fb], gsems[fb])

        return carry

    lax.fori_loop(0, NGRP, group, 0)
    pltpu.make_async_copy(rows.at[2], acc.at[didx.at[pl.ds(0, CH)]], ss2).wait()

    @pl.when(wid < NEXTRA)
    def _():
        eb = (NW * NCH + wid) * CH
        pltpu.sync_copy(ef_hbm.at[0, pl.ds(eb, CH)], sidx.at[pl.ds(0, CH)])
        pltpu.sync_copy(ef_hbm.at[1, pl.ds(eb, CH)], didx.at[pl.ds(0, CH)])
        pltpu.async_copy(y_hbm.at[sidx.at[pl.ds(0, CH)]], rows.at[0], gs0).wait()
        pltpu.sync_copy(rows.at[0], acc.at[didx.at[pl.ds(0, CH)]], add=True)

    plsc.subcore_barrier()
    pltpu.sync_copy(acc.at[pl.ds(s * RPT, RPT)],
                    out_hbm.at[c, pl.ds(s * RPT, RPT)])


def _scale_body(deg_ref, x_ref, y_ref, dinv_ref):
    dinv = lax.rsqrt(deg_ref[...])
    dinv_ref[...] = dinv
    y_ref[...] = x_ref[...] * dinv


_scale_call = pl.pallas_call(
    _scale_body,
    out_shape=(
        jax.ShapeDtypeStruct((N, D), jnp.float32),
        jax.ShapeDtypeStruct((N, 1), jnp.float32),
    ),
)


def _bn(h, g, b):
    mu = jnp.mean(h, axis=0, keepdims=True)
    var = jnp.mean((h - mu) ** 2, axis=0, keepdims=True)
    return (h - mu) * lax.rsqrt(var + EPS) * g + b


def _dense_body(part_ref, y_ref, dinv_ref, x_ref, W_ref, b_ref,
                bn_g_ref, bn_b_ref, bn1_g_ref, bn1_b_ref, W1_ref, b1_ref,
                W2_ref, b2_ref, bn2_g_ref, bn2_b_ref, out_ref):
    p0 = lax.slice(part_ref[0], (0, 0), (N, D))
    p1 = lax.slice(part_ref[1], (0, 0), (N, D))
    agg = (p0 + p1 + y_ref[...]) * dinv_ref[...]
    conv = jnp.dot(agg, W_ref[...], preferred_element_type=jnp.float32)
    conv = conv + b_ref[...]
    h = _bn(conv, bn_g_ref[...], bn_b_ref[...])
    h = jnp.maximum(h, 0.0) + x_ref[...]
    z = _bn(h, bn1_g_ref[...], bn1_b_ref[...])
    z = jnp.dot(z, W1_ref[...], preferred_element_type=jnp.float32) + b1_ref[...]
    z = jnp.maximum(z, 0.0)
    z = jnp.dot(z, W2_ref[...], preferred_element_type=jnp.float32) + b2_ref[...]
    h = h + z
    out_ref[...] = _bn(h, bn2_g_ref[...], bn2_b_ref[...])


_dense_call = pl.pallas_call(
    _dense_body,
    out_shape=jax.ShapeDtypeStruct((N, D), jnp.float32),
)


def kernel(x, edge_index, W, b, bn_g, bn_b, bn1_g, bn1_b,
           W1, b1, W2, b2, bn2_g, bn2_b):
    ef = edge_index
    zeros_vec = jnp.zeros((DEG_CH,), jnp.float32)
    zeros_mat = jnp.zeros((RPT, D), jnp.float32)

    degp = _deg_kernel(ef, zeros_vec)
    deg_col = (degp[:N] + degp[NPAD:NPAD + N] + 1.0).reshape(N, 1)
    y, dinv = _scale_call(deg_col, x)
    part = _agg_kernel(y, ef, zeros_mat)
    out = _dense_call(
        part, y, dinv, x, W, b.reshape(1, D),
        bn_g.reshape(1, D), bn_b.reshape(1, D),
        bn1_g.reshape(1, D), bn1_b.reshape(1, D),
        W1, b1.reshape(1, DFF), W2, b2.reshape(1, D),
        bn2_g.reshape(1, D), bn2_b.reshape(1, D),
    )
    return out
